# baseline (speedup 1.0000x reference)
import jax
import jax.numpy as jnp
from jax.experimental import pallas as pl
from jax.experimental.pallas import tpu as pltpu

B, T, I, H = 64, 1024, 256, 512
NCORES = 1
BC = B // NCORES
CHUNK = 128
NC = T // CHUNK
UNROLL = 32


def _rnn_body(x_ref, wih0_ref, whh0_ref, wih1_ref, whh1_ref, b0_ref, b1_ref,
              fcw_ref, fcb_ref, out_ref, h1_ref, h2_ref, xp_ref, u2p_ref,
              h1buf_ref):
    c = pl.program_id(1)

    @pl.when(c == 0)
    def _init():
        h1_ref[...] = jnp.zeros_like(h1_ref)
        h2_ref[...] = jnp.zeros_like(h2_ref)

    h1flat = h1buf_ref[...].reshape(CHUNK * BC, H)
    u2p = jnp.dot(h1flat, wih1_ref[...], preferred_element_type=jnp.float32)
    u2p_ref[...] = (u2p + b1_ref[...]).astype(jnp.bfloat16).reshape(CHUNK, BC, H)

    xb = x_ref[...].reshape(CHUNK * BC, I)
    xp = jnp.dot(xb, wih0_ref[...], preferred_element_type=jnp.float32)
    xp_ref[...] = (xp + b0_ref[...]).astype(jnp.bfloat16).reshape(CHUNK, BC, H)

    not_first = c > 0

    def step(i, carry):
        h1b, h2b = carry
        u1 = xp_ref[i].astype(jnp.float32) + jnp.dot(h1b, whh0_ref[...],
                                                     preferred_element_type=jnp.float32)
        u2 = u2p_ref[i].astype(jnp.float32) + jnp.dot(h2b, whh1_ref[...],
                                                      preferred_element_type=jnp.float32)
        h1b_new = jnp.tanh(u1).astype(jnp.bfloat16)
        h1buf_ref[i] = h1b_new
        h2b_new = jnp.where(not_first, jnp.tanh(u2),
                            jnp.float32(0.0)).astype(jnp.bfloat16)
        return h1b_new, h2b_new

    h1b, h2b = jax.lax.fori_loop(0, CHUNK, step, (h1_ref[...], h2_ref[...]),
                                 unroll=UNROLL)
    h1_ref[...] = h1b
    h2_ref[...] = h2b

    @pl.when(c == NC - 1)
    def _tail():
        h1flat_last = h1buf_ref[...].reshape(CHUNK * BC, H)
        u2t = jnp.dot(h1flat_last, wih1_ref[...],
                      preferred_element_type=jnp.float32)
        u2p_ref[...] = (u2t + b1_ref[...]).astype(jnp.bfloat16).reshape(CHUNK, BC, H)

        def step2(i, h2c):
            u2 = u2p_ref[i].astype(jnp.float32) + jnp.dot(h2c, whh1_ref[...],
                                                          preferred_element_type=jnp.float32)
            return jnp.tanh(u2).astype(jnp.bfloat16)

        h2fin = jax.lax.fori_loop(0, CHUNK, step2, h2b, unroll=UNROLL)
        out_ref[...] = jnp.dot(h2fin, fcw_ref[...],
                               preferred_element_type=jnp.float32) + fcb_ref[...]


def kernel(x, W_ih0, W_hh0, b_ih0, b_hh0, W_ih1, W_hh1, b_ih1, b_hh1, fc_W, fc_b):
    xT = jnp.swapaxes(x, 0, 1).astype(jnp.bfloat16)
    wih0 = W_ih0.T.astype(jnp.bfloat16)
    whh0 = W_hh0.T.astype(jnp.bfloat16)
    wih1 = W_ih1.T.astype(jnp.bfloat16)
    whh1 = W_hh1.T.astype(jnp.bfloat16)
    b0 = (b_ih0 + b_hh0).reshape(1, H)
    b1 = (b_ih1 + b_hh1).reshape(1, H)
    fcw = fc_W.T.astype(jnp.bfloat16)
    fcb = fc_b.reshape(1, 2)

    grid = (NCORES, NC)
    out = pl.pallas_call(
        _rnn_body,
        grid=grid,
        in_specs=[
            pl.BlockSpec((CHUNK, BC, I), lambda b, c: (c, b, 0)),
            pl.BlockSpec((I, H), lambda b, c: (0, 0)),
            pl.BlockSpec((H, H), lambda b, c: (0, 0)),
            pl.BlockSpec((H, H), lambda b, c: (0, 0)),
            pl.BlockSpec((H, H), lambda b, c: (0, 0)),
            pl.BlockSpec((1, H), lambda b, c: (0, 0)),
            pl.BlockSpec((1, H), lambda b, c: (0, 0)),
            pl.BlockSpec((H, 2), lambda b, c: (0, 0)),
            pl.BlockSpec((1, 2), lambda b, c: (0, 0)),
        ],
        out_specs=pl.BlockSpec((BC, 2), lambda b, c: (b, 0)),
        out_shape=jax.ShapeDtypeStruct((B, 2), jnp.float32),
        scratch_shapes=[
            pltpu.VMEM((BC, H), jnp.bfloat16),
            pltpu.VMEM((BC, H), jnp.bfloat16),
            pltpu.VMEM((CHUNK, BC, H), jnp.bfloat16),
            pltpu.VMEM((CHUNK, BC, H), jnp.bfloat16),
            pltpu.VMEM((CHUNK, BC, H), jnp.bfloat16),
        ],
        compiler_params=pltpu.CompilerParams(
            dimension_semantics=("parallel", "arbitrary"),
            vmem_limit_bytes=63 * 1024 * 1024,
        ),
    )(xT, wih0, whh0, wih1, whh1, b0, b1, fcw, fcb)
    return out


# speedup vs baseline: 1.0543x; 1.0543x over previous
import jax
import jax.numpy as jnp
from jax.experimental import pallas as pl
from jax.experimental.pallas import tpu as pltpu

B, T, I, H = 64, 1024, 256, 512
NCORES = 1
BC = B // NCORES
CHUNK = 64
NC = T // CHUNK
UNROLL = 64


def _rnn_body(x_ref, wih0_ref, whh0_ref, wih1_ref, whh1_ref, b0_ref, b1_ref,
              fcw_ref, fcb_ref, out_ref, h1_ref, h2_ref, xp_ref, u2p_ref,
              h1buf_ref):
    c = pl.program_id(1)

    @pl.when(c == 0)
    def _init():
        h1_ref[...] = jnp.zeros_like(h1_ref)
        h2_ref[...] = jnp.zeros_like(h2_ref)

    h1flat = h1buf_ref[...].reshape(CHUNK * BC, H)
    u2p = jnp.dot(h1flat, wih1_ref[...], preferred_element_type=jnp.float32)
    u2p_ref[...] = (u2p + b1_ref[...]).astype(jnp.bfloat16).reshape(CHUNK, BC, H)

    xb = x_ref[...].reshape(CHUNK * BC, I)
    xp = jnp.dot(xb, wih0_ref[...], preferred_element_type=jnp.float32)
    xp_ref[...] = (xp + b0_ref[...]).astype(jnp.bfloat16).reshape(CHUNK, BC, H)

    not_first = c > 0

    def step(i, carry):
        h1b, h2b = carry
        u1 = xp_ref[i].astype(jnp.float32) + jnp.dot(h1b, whh0_ref[...],
                                                     preferred_element_type=jnp.float32)
        u2 = u2p_ref[i].astype(jnp.float32) + jnp.dot(h2b, whh1_ref[...],
                                                      preferred_element_type=jnp.float32)
        h1b_new = jnp.tanh(u1).astype(jnp.bfloat16)
        h1buf_ref[i] = h1b_new
        h2b_new = jnp.where(not_first, jnp.tanh(u2),
                            jnp.float32(0.0)).astype(jnp.bfloat16)
        return h1b_new, h2b_new

    h1b, h2b = jax.lax.fori_loop(0, CHUNK, step, (h1_ref[...], h2_ref[...]),
                                 unroll=UNROLL)
    h1_ref[...] = h1b
    h2_ref[...] = h2b

    @pl.when(c == NC - 1)
    def _tail():
        h1flat_last = h1buf_ref[...].reshape(CHUNK * BC, H)
        u2t = jnp.dot(h1flat_last, wih1_ref[...],
                      preferred_element_type=jnp.float32)
        u2p_ref[...] = (u2t + b1_ref[...]).astype(jnp.bfloat16).reshape(CHUNK, BC, H)

        def step2(i, h2c):
            u2 = u2p_ref[i].astype(jnp.float32) + jnp.dot(h2c, whh1_ref[...],
                                                          preferred_element_type=jnp.float32)
            return jnp.tanh(u2).astype(jnp.bfloat16)

        h2fin = jax.lax.fori_loop(0, CHUNK, step2, h2b, unroll=UNROLL)
        out_ref[...] = jnp.dot(h2fin, fcw_ref[...],
                               preferred_element_type=jnp.float32) + fcb_ref[...]


def kernel(x, W_ih0, W_hh0, b_ih0, b_hh0, W_ih1, W_hh1, b_ih1, b_hh1, fc_W, fc_b):
    xT = jnp.swapaxes(x, 0, 1).astype(jnp.bfloat16)
    wih0 = W_ih0.T.astype(jnp.bfloat16)
    whh0 = W_hh0.T.astype(jnp.bfloat16)
    wih1 = W_ih1.T.astype(jnp.bfloat16)
    whh1 = W_hh1.T.astype(jnp.bfloat16)
    b0 = (b_ih0 + b_hh0).reshape(1, H)
    b1 = (b_ih1 + b_hh1).reshape(1, H)
    fcw = fc_W.T.astype(jnp.bfloat16)
    fcb = fc_b.reshape(1, 2)

    grid = (NCORES, NC)
    out = pl.pallas_call(
        _rnn_body,
        grid=grid,
        in_specs=[
            pl.BlockSpec((CHUNK, BC, I), lambda b, c: (c, b, 0)),
            pl.BlockSpec((I, H), lambda b, c: (0, 0)),
            pl.BlockSpec((H, H), lambda b, c: (0, 0)),
            pl.BlockSpec((H, H), lambda b, c: (0, 0)),
            pl.BlockSpec((H, H), lambda b, c: (0, 0)),
            pl.BlockSpec((1, H), lambda b, c: (0, 0)),
            pl.BlockSpec((1, H), lambda b, c: (0, 0)),
            pl.BlockSpec((H, 2), lambda b, c: (0, 0)),
            pl.BlockSpec((1, 2), lambda b, c: (0, 0)),
        ],
        out_specs=pl.BlockSpec((BC, 2), lambda b, c: (b, 0)),
        out_shape=jax.ShapeDtypeStruct((B, 2), jnp.float32),
        scratch_shapes=[
            pltpu.VMEM((BC, H), jnp.bfloat16),
            pltpu.VMEM((BC, H), jnp.bfloat16),
            pltpu.VMEM((CHUNK, BC, H), jnp.bfloat16),
            pltpu.VMEM((CHUNK, BC, H), jnp.bfloat16),
            pltpu.VMEM((CHUNK, BC, H), jnp.bfloat16),
        ],
        compiler_params=pltpu.CompilerParams(
            dimension_semantics=("parallel", "arbitrary"),
            vmem_limit_bytes=63 * 1024 * 1024,
        ),
    )(xT, wih0, whh0, wih1, whh1, b0, b1, fcw, fcb)
    return out
